# baseline (speedup 1.0000x reference)
import jax
import jax.numpy as jnp
from jax import lax
from jax.experimental import pallas as pl
from jax.experimental.pallas import tpu as pltpu
from jax.experimental.pallas import tpu_sc as plsc

EMBED = 32
TBL = 1280
NCOORD = 4
LANES = 16
NCORES = 2
NSUBCORES = 16
NWORKERS = NCORES * NSUBCORES

CHUNK = 1280
SUB = 128
NSUB = CHUNK // SUB
GRP = CHUNK // LANES
NBUF = 2


def _body(boxes_hbm, table_hbm, out_hbm, box_v, *scr):
    t_dim, n_btile, _, _ = boxes_hbm.shape
    bat_per_w = (n_btile * 128) // NWORKERS
    per_w = bat_per_w * t_dim * NCOORD
    nchunk = per_w // CHUNK
    idxv = scr[0:NBUF]
    rowv = scr[NBUF:2 * NBUF]
    table_sh = scr[2 * NBUF]
    gsem = scr[2 * NBUF + 1:3 * NBUF + 1]
    osem = scr[3 * NBUF + 1:4 * NBUF + 1]

    wid = lax.axis_index("s") * NCORES + lax.axis_index("c")
    base_w = wid * per_w
    btile = wid // (128 // bat_per_w)
    lane0 = (wid % (128 // bat_per_w)) * bat_per_w

    iota = lax.iota(jnp.int32, LANES)
    c_idx = iota % NCOORD
    tq_idx = iota // NCOORD
    offs = c_idx * TBL

    sid = lax.axis_index("s")
    tbl_rows = NCOORD * TBL // NSUBCORES
    pltpu.sync_copy(table_hbm.at[pl.ds(sid * tbl_rows, tbl_rows)],
                    table_sh.at[pl.ds(sid * tbl_rows, tbl_rows)])
    pltpu.sync_copy(boxes_hbm.at[:, btile, :, pl.ds(lane0, 32)], box_v)
    plsc.subcore_barrier()

    def compute_idx(g, b):
        gi0 = g * GRP

        def cvt(si, _):
            gi = gi0 + si
            bt = gi * NCOORD
            b_loc = bt // t_dim
            t0 = bt % t_dim
            v = plsc.load_gather(
                box_v, (t0 + tq_idx, c_idx, jnp.full((LANES,), b_loc, jnp.int32)))
            v = jnp.minimum(jnp.maximum(v, 0.0), float(TBL - 1))
            idxv[b][pl.ds(si * LANES, LANES)] = v.astype(jnp.int32) + offs
            return 0

        lax.fori_loop(0, GRP, cvt, 0)

    def fire_gathers(b):
        for j in range(NSUB):
            sl = pl.ds(j * SUB, SUB)
            pltpu.async_copy(table_sh.at[idxv[b].at[sl]], rowv[b].at[sl], gsem[b])

    def wait_writeback(b):
        pltpu.make_async_copy(rowv[b], out_hbm.at[pl.ds(base_w, CHUNK)], osem[b]).wait()

    def drain_gathers(b):
        pltpu.make_async_copy(out_hbm.at[pl.ds(base_w, CHUNK)], rowv[b], gsem[b]).wait()

    def step(g, b, nb, prep, first_ring):
        if prep:
            compute_idx(g + 1, nb)
            if first_ring:
                @pl.when(g >= NBUF - 1)
                def _w():
                    wait_writeback(nb)
            else:
                wait_writeback(nb)
            fire_gathers(nb)
        drain_gathers(b)
        pltpu.async_copy(rowv[b], out_hbm.at[pl.ds(base_w + g * CHUNK, CHUNK)],
                         osem[b])

    compute_idx(0, 0)
    fire_gathers(0)

    def ring(p, _):
        for b in range(NBUF):
            g = p * NBUF + b
            step(g, b, (b + 1) % NBUF, True, True)
        return 0

    nring = (nchunk - 1) // NBUF
    lax.fori_loop(0, nring, ring, 0)
    for g in range(nring * NBUF, nchunk):
        step(g, g % NBUF, (g + 1) % NBUF, g + 1 < nchunk, False)
    for b in range(NBUF):
        wait_writeback(b)


@jax.jit
def _gather_call(boxes_phys, table):
    t_dim, n_btile, _, _ = boxes_phys.shape
    n = t_dim * n_btile * 128 * NCOORD
    mesh = plsc.VectorSubcoreMesh(core_axis_name="c", subcore_axis_name="s")
    return pl.kernel(
        _body,
        out_type=jax.ShapeDtypeStruct((n, EMBED), jnp.float32),
        mesh=mesh,
        scratch_types=(
            [pltpu.VMEM((t_dim, NCOORD, 32), jnp.float32)]
            + [pltpu.VMEM((CHUNK,), jnp.int32)] * NBUF
            + [pltpu.VMEM((CHUNK, EMBED), jnp.float32)] * NBUF
            + [pltpu.VMEM_SHARED((NCOORD * TBL, EMBED), jnp.float32)]
            + [pltpu.SemaphoreType.DMA] * (2 * NBUF)
        ),
        compiler_params=pltpu.CompilerParams(use_tc_tiling_on_sc=False,
                                             needs_layout_passes=False),
    )(boxes_phys, table)


def kernel(boxes, x_table, y_table, w_table, h_table):
    b, t, _ = boxes.shape
    table = jnp.concatenate([x_table, y_table, w_table, h_table], axis=0)
    boxes_phys = (boxes.transpose(1, 2, 0)
                  .reshape(t, NCOORD, b // 128, 128)
                  .transpose(0, 2, 1, 3))
    out = _gather_call(boxes_phys, table)
    return out.reshape(b, t, NCOORD * EMBED)


# speedup vs baseline: 1.0622x; 1.0622x over previous
import jax
import jax.numpy as jnp
from jax import lax
from jax.experimental import pallas as pl
from jax.experimental.pallas import tpu as pltpu
from jax.experimental.pallas import tpu_sc as plsc

EMBED = 32
TBL = 1280
NCOORD = 4
LANES = 16
NCORES = 2
NSUBCORES = 16
NWORKERS = NCORES * NSUBCORES

CHUNK = 640
SUB = 128
NSUB = CHUNK // SUB
GRP = CHUNK // LANES
NBUF = 3


def _body(boxes_hbm, table_hbm, out_hbm, box_v, *scr):
    t_dim, n_btile, _, _ = boxes_hbm.shape
    bat_per_w = (n_btile * 128) // NWORKERS
    per_w = bat_per_w * t_dim * NCOORD
    nchunk = per_w // CHUNK
    idxv = scr[0:NBUF]
    rowv = scr[NBUF:2 * NBUF]
    table_sh = scr[2 * NBUF]
    gsem = scr[2 * NBUF + 1:3 * NBUF + 1]
    osem = scr[3 * NBUF + 1:4 * NBUF + 1]

    wid = lax.axis_index("s") * NCORES + lax.axis_index("c")
    base_w = wid * per_w
    btile = wid // (128 // bat_per_w)
    lane0 = (wid % (128 // bat_per_w)) * bat_per_w

    iota = lax.iota(jnp.int32, LANES)
    c_idx = iota % NCOORD
    tq_idx = iota // NCOORD
    offs = c_idx * TBL

    sid = lax.axis_index("s")
    tbl_rows = NCOORD * TBL // NSUBCORES
    pltpu.sync_copy(table_hbm.at[pl.ds(sid * tbl_rows, tbl_rows)],
                    table_sh.at[pl.ds(sid * tbl_rows, tbl_rows)])
    pltpu.sync_copy(boxes_hbm.at[:, btile, :, pl.ds(lane0, 32)], box_v)
    plsc.subcore_barrier()

    def compute_idx(g, b):
        gi0 = g * GRP

        def cvt(si, _):
            gi = gi0 + si
            bt = gi * NCOORD
            b_loc = bt // t_dim
            t0 = bt % t_dim
            v = plsc.load_gather(
                box_v, (t0 + tq_idx, c_idx, jnp.full((LANES,), b_loc, jnp.int32)))
            v = jnp.minimum(jnp.maximum(v, 0.0), float(TBL - 1))
            idxv[b][pl.ds(si * LANES, LANES)] = v.astype(jnp.int32) + offs
            return 0

        lax.fori_loop(0, GRP, cvt, 0)

    def fire_gathers(b):
        for j in range(NSUB):
            sl = pl.ds(j * SUB, SUB)
            pltpu.async_copy(table_sh.at[idxv[b].at[sl]], rowv[b].at[sl], gsem[b])

    def wait_writeback(b):
        pltpu.make_async_copy(rowv[b], out_hbm.at[pl.ds(base_w, CHUNK)], osem[b]).wait()

    def drain_gathers(b):
        pltpu.make_async_copy(out_hbm.at[pl.ds(base_w, CHUNK)], rowv[b], gsem[b]).wait()

    def step(g, b, nb, prep, first_ring):
        if prep:
            compute_idx(g + 1, nb)
            if first_ring:
                @pl.when(g >= NBUF - 1)
                def _w():
                    wait_writeback(nb)
            else:
                wait_writeback(nb)
            fire_gathers(nb)
        drain_gathers(b)
        pltpu.async_copy(rowv[b], out_hbm.at[pl.ds(base_w + g * CHUNK, CHUNK)],
                         osem[b])

    compute_idx(0, 0)
    fire_gathers(0)

    def ring(p, _):
        for b in range(NBUF):
            g = p * NBUF + b
            step(g, b, (b + 1) % NBUF, True, True)
        return 0

    nring = (nchunk - 1) // NBUF
    lax.fori_loop(0, nring, ring, 0)
    for g in range(nring * NBUF, nchunk):
        step(g, g % NBUF, (g + 1) % NBUF, g + 1 < nchunk, False)
    for b in range(NBUF):
        wait_writeback(b)


@jax.jit
def _gather_call(boxes_phys, table):
    t_dim, n_btile, _, _ = boxes_phys.shape
    n = t_dim * n_btile * 128 * NCOORD
    mesh = plsc.VectorSubcoreMesh(core_axis_name="c", subcore_axis_name="s")
    return pl.kernel(
        _body,
        out_type=jax.ShapeDtypeStruct((n, EMBED), jnp.float32),
        mesh=mesh,
        scratch_types=(
            [pltpu.VMEM((t_dim, NCOORD, 32), jnp.float32)]
            + [pltpu.VMEM((CHUNK,), jnp.int32)] * NBUF
            + [pltpu.VMEM((CHUNK, EMBED), jnp.float32)] * NBUF
            + [pltpu.VMEM_SHARED((NCOORD * TBL, EMBED), jnp.float32)]
            + [pltpu.SemaphoreType.DMA] * (2 * NBUF)
        ),
        compiler_params=pltpu.CompilerParams(use_tc_tiling_on_sc=False,
                                             needs_layout_passes=False),
    )(boxes_phys, table)


def kernel(boxes, x_table, y_table, w_table, h_table):
    b, t, _ = boxes.shape
    table = jnp.concatenate([x_table, y_table, w_table, h_table], axis=0)
    boxes_phys = (boxes.transpose(1, 2, 0)
                  .reshape(t, NCOORD, b // 128, 128)
                  .transpose(0, 2, 1, 3))
    out = _gather_call(boxes_phys, table)
    return out.reshape(b, t, NCOORD * EMBED)


# speedup vs baseline: 1.0820x; 1.0186x over previous
import jax
import jax.numpy as jnp
from jax import lax
from jax.experimental import pallas as pl
from jax.experimental.pallas import tpu as pltpu
from jax.experimental.pallas import tpu_sc as plsc

EMBED = 32
TBL = 1280
NCOORD = 4
LANES = 16
NCORES = 2
NSUBCORES = 16
NWORKERS = NCORES * NSUBCORES

CHUNK = 512
SUB = 128
NSUB = CHUNK // SUB
GRP = CHUNK // LANES
NBUF = 3


def _body(boxes_hbm, table_hbm, out_hbm, box_v, *scr):
    t_dim, n_btile, _, _ = boxes_hbm.shape
    bat_per_w = (n_btile * 128) // NWORKERS
    per_w = bat_per_w * t_dim * NCOORD
    nchunk = per_w // CHUNK
    idxv = scr[0:NBUF]
    rowv = scr[NBUF:2 * NBUF]
    table_sh = scr[2 * NBUF]
    gsem = scr[2 * NBUF + 1:3 * NBUF + 1]
    osem = scr[3 * NBUF + 1:4 * NBUF + 1]

    wid = lax.axis_index("s") * NCORES + lax.axis_index("c")
    base_w = wid * per_w
    btile = wid // (128 // bat_per_w)
    lane0 = (wid % (128 // bat_per_w)) * bat_per_w

    iota = lax.iota(jnp.int32, LANES)
    c_idx = iota % NCOORD
    tq_idx = iota // NCOORD
    offs = c_idx * TBL

    sid = lax.axis_index("s")
    tbl_rows = NCOORD * TBL // NSUBCORES
    pltpu.sync_copy(table_hbm.at[pl.ds(sid * tbl_rows, tbl_rows)],
                    table_sh.at[pl.ds(sid * tbl_rows, tbl_rows)])
    pltpu.sync_copy(boxes_hbm.at[:, btile, :, pl.ds(lane0, 32)], box_v)
    plsc.subcore_barrier()

    def compute_idx(g, b):
        gi0 = g * GRP

        def cvt(si, _):
            gi = gi0 + si
            bt = gi * NCOORD
            b_loc = bt // t_dim
            t0 = bt % t_dim
            v = plsc.load_gather(
                box_v, (t0 + tq_idx, c_idx, jnp.full((LANES,), b_loc, jnp.int32)))
            v = jnp.minimum(jnp.maximum(v, 0.0), float(TBL - 1))
            idxv[b][pl.ds(si * LANES, LANES)] = v.astype(jnp.int32) + offs
            return 0

        lax.fori_loop(0, GRP, cvt, 0)

    def fire_gathers(b):
        for j in range(NSUB):
            sl = pl.ds(j * SUB, SUB)
            pltpu.async_copy(table_sh.at[idxv[b].at[sl]], rowv[b].at[sl], gsem[b])

    def wait_writeback(b):
        pltpu.make_async_copy(rowv[b], out_hbm.at[pl.ds(base_w, CHUNK)], osem[b]).wait()

    def drain_gathers(b):
        pltpu.make_async_copy(out_hbm.at[pl.ds(base_w, CHUNK)], rowv[b], gsem[b]).wait()

    def step(g, b, nb, prep, first_ring):
        if prep:
            compute_idx(g + 1, nb)
            if first_ring:
                @pl.when(g >= NBUF - 1)
                def _w():
                    wait_writeback(nb)
            else:
                wait_writeback(nb)
            fire_gathers(nb)
        drain_gathers(b)
        pltpu.async_copy(rowv[b], out_hbm.at[pl.ds(base_w + g * CHUNK, CHUNK)],
                         osem[b])

    compute_idx(0, 0)
    fire_gathers(0)

    def ring(p, _):
        for b in range(NBUF):
            g = p * NBUF + b
            step(g, b, (b + 1) % NBUF, True, True)
        return 0

    nring = (nchunk - 1) // NBUF
    lax.fori_loop(0, nring, ring, 0)
    for g in range(nring * NBUF, nchunk):
        step(g, g % NBUF, (g + 1) % NBUF, g + 1 < nchunk, False)
    for b in range(NBUF):
        wait_writeback(b)


@jax.jit
def _gather_call(boxes_phys, table):
    t_dim, n_btile, _, _ = boxes_phys.shape
    n = t_dim * n_btile * 128 * NCOORD
    mesh = plsc.VectorSubcoreMesh(core_axis_name="c", subcore_axis_name="s")
    return pl.kernel(
        _body,
        out_type=jax.ShapeDtypeStruct((n, EMBED), jnp.float32),
        mesh=mesh,
        scratch_types=(
            [pltpu.VMEM((t_dim, NCOORD, 32), jnp.float32)]
            + [pltpu.VMEM((CHUNK,), jnp.int32)] * NBUF
            + [pltpu.VMEM((CHUNK, EMBED), jnp.float32)] * NBUF
            + [pltpu.VMEM_SHARED((NCOORD * TBL, EMBED), jnp.float32)]
            + [pltpu.SemaphoreType.DMA] * (2 * NBUF)
        ),
        compiler_params=pltpu.CompilerParams(use_tc_tiling_on_sc=False,
                                             needs_layout_passes=False),
    )(boxes_phys, table)


def kernel(boxes, x_table, y_table, w_table, h_table):
    b, t, _ = boxes.shape
    table = jnp.concatenate([x_table, y_table, w_table, h_table], axis=0)
    boxes_phys = (boxes.transpose(1, 2, 0)
                  .reshape(t, NCOORD, b // 128, 128)
                  .transpose(0, 2, 1, 3))
    out = _gather_call(boxes_phys, table)
    return out.reshape(b, t, NCOORD * EMBED)


# speedup vs baseline: 1.1366x; 1.0505x over previous
import jax
import jax.numpy as jnp
from jax import lax
from jax.experimental import pallas as pl
from jax.experimental.pallas import tpu as pltpu
from jax.experimental.pallas import tpu_sc as plsc

EMBED = 32
TBL = 1280
NCOORD = 4
LANES = 16
NCORES = 2
NSUBCORES = 16
NWORKERS = NCORES * NSUBCORES

CHUNK = 256
SUB = 128
NSUB = CHUNK // SUB
GRP = CHUNK // LANES
NBUF = 3


def _body(boxes_hbm, table_hbm, out_hbm, box_v, *scr):
    t_dim, n_btile, _, _ = boxes_hbm.shape
    bat_per_w = (n_btile * 128) // NWORKERS
    per_w = bat_per_w * t_dim * NCOORD
    nchunk = per_w // CHUNK
    idxv = scr[0:NBUF]
    rowv = scr[NBUF:2 * NBUF]
    table_sh = scr[2 * NBUF]
    gsem = scr[2 * NBUF + 1:3 * NBUF + 1]
    osem = scr[3 * NBUF + 1:4 * NBUF + 1]

    wid = lax.axis_index("s") * NCORES + lax.axis_index("c")
    base_w = wid * per_w
    btile = wid // (128 // bat_per_w)
    lane0 = (wid % (128 // bat_per_w)) * bat_per_w

    iota = lax.iota(jnp.int32, LANES)
    c_idx = iota % NCOORD
    tq_idx = iota // NCOORD
    offs = c_idx * TBL

    sid = lax.axis_index("s")
    tbl_rows = NCOORD * TBL // NSUBCORES
    pltpu.sync_copy(table_hbm.at[pl.ds(sid * tbl_rows, tbl_rows)],
                    table_sh.at[pl.ds(sid * tbl_rows, tbl_rows)])
    pltpu.sync_copy(boxes_hbm.at[:, btile, :, pl.ds(lane0, 32)], box_v)
    plsc.subcore_barrier()

    def compute_idx(g, b):
        gi0 = g * GRP

        def cvt(si, _):
            gi = gi0 + si
            bt = gi * NCOORD
            b_loc = bt // t_dim
            t0 = bt % t_dim
            v = plsc.load_gather(
                box_v, (t0 + tq_idx, c_idx, jnp.full((LANES,), b_loc, jnp.int32)))
            v = jnp.minimum(jnp.maximum(v, 0.0), float(TBL - 1))
            idxv[b][pl.ds(si * LANES, LANES)] = v.astype(jnp.int32) + offs
            return 0

        lax.fori_loop(0, GRP, cvt, 0)

    def fire_gathers(b):
        for j in range(NSUB):
            sl = pl.ds(j * SUB, SUB)
            pltpu.async_copy(table_sh.at[idxv[b].at[sl]], rowv[b].at[sl], gsem[b])

    def wait_writeback(b):
        pltpu.make_async_copy(rowv[b], out_hbm.at[pl.ds(base_w, CHUNK)], osem[b]).wait()

    def drain_gathers(b):
        pltpu.make_async_copy(out_hbm.at[pl.ds(base_w, CHUNK)], rowv[b], gsem[b]).wait()

    def step(g, b, nb, prep, first_ring):
        if prep:
            compute_idx(g + 1, nb)
            if first_ring:
                @pl.when(g >= NBUF - 1)
                def _w():
                    wait_writeback(nb)
            else:
                wait_writeback(nb)
            fire_gathers(nb)
        drain_gathers(b)
        pltpu.async_copy(rowv[b], out_hbm.at[pl.ds(base_w + g * CHUNK, CHUNK)],
                         osem[b])

    compute_idx(0, 0)
    fire_gathers(0)

    def ring(p, _):
        for b in range(NBUF):
            g = p * NBUF + b
            step(g, b, (b + 1) % NBUF, True, True)
        return 0

    nring = (nchunk - 1) // NBUF
    lax.fori_loop(0, nring, ring, 0)
    for g in range(nring * NBUF, nchunk):
        step(g, g % NBUF, (g + 1) % NBUF, g + 1 < nchunk, False)
    for b in range(NBUF):
        wait_writeback(b)


@jax.jit
def _gather_call(boxes_phys, table):
    t_dim, n_btile, _, _ = boxes_phys.shape
    n = t_dim * n_btile * 128 * NCOORD
    mesh = plsc.VectorSubcoreMesh(core_axis_name="c", subcore_axis_name="s")
    return pl.kernel(
        _body,
        out_type=jax.ShapeDtypeStruct((n, EMBED), jnp.float32),
        mesh=mesh,
        scratch_types=(
            [pltpu.VMEM((t_dim, NCOORD, 32), jnp.float32)]
            + [pltpu.VMEM((CHUNK,), jnp.int32)] * NBUF
            + [pltpu.VMEM((CHUNK, EMBED), jnp.float32)] * NBUF
            + [pltpu.VMEM_SHARED((NCOORD * TBL, EMBED), jnp.float32)]
            + [pltpu.SemaphoreType.DMA] * (2 * NBUF)
        ),
        compiler_params=pltpu.CompilerParams(use_tc_tiling_on_sc=False,
                                             needs_layout_passes=False),
    )(boxes_phys, table)


def kernel(boxes, x_table, y_table, w_table, h_table):
    b, t, _ = boxes.shape
    table = jnp.concatenate([x_table, y_table, w_table, h_table], axis=0)
    boxes_phys = (boxes.transpose(1, 2, 0)
                  .reshape(t, NCOORD, b // 128, 128)
                  .transpose(0, 2, 1, 3))
    out = _gather_call(boxes_phys, table)
    return out.reshape(b, t, NCOORD * EMBED)
